# Initial kernel scaffold; baseline (speedup 1.0000x reference)
import jax
import jax.numpy as jnp
from jax.experimental import pallas as pl


def kernel(x, edge_index, W0, a_src0, a_dst0, b0, W1, a_src1, a_dst1, b1):
    raise NotImplementedError("write your pallas kernel here")


# speedup vs baseline: 57.5999x; 57.5999x over previous
import functools

import jax
import jax.numpy as jnp
from jax import lax
from jax.experimental import pallas as pl
from jax.experimental.pallas import tpu as pltpu
from jax.experimental.pallas import tpu_sc as plsc

NEG = -1.0e30
BIG = 1.0e30
B = 128
ZR = 32
BLK = 512


def _tc_prep0(x_ref, w0_ref, as_ref, ad_ref, n_ref, h0_o, u_o, v_o, g_o, gsc):
    i = pl.program_id(0)
    n_real = n_ref[0]
    h0 = jnp.dot(x_ref[...], w0_ref[...], preferred_element_type=jnp.float32)
    h0_o[...] = h0
    ps = h0 * as_ref[...]
    pd = h0 * ad_ref[...]
    H = u_o.shape[1] // 2
    C = h0.shape[1] // H
    asrc = jnp.concatenate(
        [jnp.sum(ps[:, h * C:(h + 1) * C], axis=1, keepdims=True) for h in range(H)], axis=1)
    adst = jnp.concatenate(
        [jnp.sum(pd[:, h * C:(h + 1) * C], axis=1, keepdims=True) for h in range(H)], axis=1)
    rowid = i * BLK + lax.broadcasted_iota(jnp.int32, (BLK, 1), 0)
    valid = rowid < n_real
    asrc = jnp.where(valid, asrc, NEG)
    adst = jnp.where(valid, adst, NEG)
    z8 = jnp.zeros((BLK, H), jnp.float32)
    u_o[...] = jnp.concatenate([asrc, z8], axis=1)
    v_o[...] = jnp.concatenate([adst, z8], axis=1)

    @pl.when(i == 0)
    def _():
        gsc[...] = jnp.full((8, 128), NEG, jnp.float32)

    bm = jnp.max(asrc, axis=0, keepdims=True)
    gsc[0:1, 0:H] = jnp.maximum(gsc[0:1, 0:H], bm)
    g_o[...] = jnp.concatenate(
        [gsc[0:1, 0:H], jnp.full((1, 16 - H), BIG, jnp.float32)], axis=1)


def _tc_mid(a0_ref, a1_ref, wa0_ref, wa1_ref, b0_ref, w1_ref, as1_ref, ad1_ref,
            n_ref, h1e_o, adt_o, g1_o, gsc):
    i = pl.program_id(0)
    n_real = n_ref[0]
    s = a0_ref[...] + a1_ref[...]
    den = wa0_ref[...] + wa1_ref[...]
    HC = b0_ref.shape[1]
    H = wa0_ref.shape[1] // 2
    C = HC // H
    outs = []
    for h in range(H):
        outs.append(s[:, h * C:(h + 1) * C] / (den[:, h:h + 1] + 1e-16))
    z = jnp.concatenate(outs, axis=1) + b0_ref[...]
    hp = jnp.where(z > 0, z, jnp.exp(jnp.minimum(z, 0.0)) - 1.0)
    h1 = jnp.dot(hp, w1_ref[...], preferred_element_type=jnp.float32)
    as1 = jnp.sum(h1 * as1_ref[...], axis=1, keepdims=True)
    ad1 = jnp.sum(h1 * ad1_ref[...], axis=1, keepdims=True)
    rowid = i * BLK + lax.broadcasted_iota(jnp.int32, (BLK, 1), 0)
    valid = rowid < n_real
    as1 = jnp.where(valid, as1, NEG)
    ad1 = jnp.where(valid, ad1, NEG)
    OC = h1.shape[1]
    h1e_o[...] = jnp.concatenate(
        [h1, jnp.ones((BLK, 1), jnp.float32), jnp.zeros((BLK, 15), jnp.float32)], axis=1)
    adt_o[...] = jnp.concatenate(
        [as1, ad1, jnp.zeros((BLK, 14), jnp.float32)], axis=1)

    @pl.when(i == 0)
    def _():
        gsc[...] = jnp.full((8, 128), NEG, jnp.float32)

    gsc[0:1, 0:1] = jnp.maximum(gsc[0:1, 0:1], jnp.max(as1, axis=0, keepdims=True))
    g1_o[...] = jnp.broadcast_to(gsc[0:1, 0:1], (1, 16))


def _tc_final(a0_ref, a1_ref, b1_ref, out_o):
    s = a0_ref[...] + a1_ref[...]
    OC = b1_ref.shape[1]
    out_o[...] = s[:, 0:OC] / (s[:, OC:OC + 1] + 1e-16) + b1_ref[...]


def _sc_edge0(np_, chunks):
    rows_pt = np_ // 16

    def body(src_hbm, dst_hbm, u_hbm, v_hbm, h0_hbm, g_hbm, outh_hbm, outw_hbm,
             acch, accw, zbuf, zwbuf, gv, sbuf, dbuf, ubuf, vbuf, wbuf, hbuf,
             s1, s2, s3):
        cid = lax.axis_index("c")
        sid = lax.axis_index("s")
        z16 = jnp.zeros((16,), jnp.float32)

        def zrow(r, _):
            for j in range(8):
                zbuf[r, pl.ds(j * 16, 16)] = z16
            zwbuf[r] = z16
            return 0
        lax.fori_loop(0, ZR, zrow, 0)

        def zcopy(k, _):
            pltpu.sync_copy(zbuf, acch.at[pl.ds(sid * rows_pt + k * ZR, ZR)])
            pltpu.sync_copy(zwbuf, accw.at[pl.ds(sid * rows_pt + k * ZR, ZR)])
            return 0
        lax.fori_loop(0, rows_pt // ZR, zcopy, 0)
        plsc.subcore_barrier()

        pltpu.sync_copy(g_hbm, gv)
        g = gv[...]
        wid = sid * 2 + cid

        def chunk(k, _):
            base = (wid * chunks + k) * B
            pltpu.sync_copy(src_hbm.at[pl.ds(base, B)], sbuf)
            pltpu.sync_copy(dst_hbm.at[pl.ds(base, B)], dbuf)
            cu = pltpu.async_copy(u_hbm.at[sbuf], ubuf, s1)
            cv = pltpu.async_copy(v_hbm.at[dbuf], vbuf, s2)
            ch = pltpu.async_copy(h0_hbm.at[sbuf], hbuf, s3)
            cu.wait()
            cv.wait()

            def wcalc(e, _):
                uz = ubuf[e]
                vz = vbuf[e]
                zz = uz + vz
                lz = jnp.where(zz > 0, zz, 0.2 * zz)
                m = jnp.maximum(g + vz, 0.0)
                wbuf[e] = jnp.exp(lz - m)
                return 0
            lax.fori_loop(0, B, wcalc, 0)
            ch.wait()

            def mcalc(e, _):
                wvec = wbuf[e]
                for j in range(8):
                    ws = wvec[j]
                    hbuf[e, pl.ds(j * 16, 16)] = ws * hbuf[e, pl.ds(j * 16, 16)]
                return 0
            lax.fori_loop(0, B, mcalc, 0)
            pltpu.sync_copy(hbuf, acch.at[dbuf], add=True)
            pltpu.sync_copy(wbuf, accw.at[dbuf], add=True)
            return 0
        lax.fori_loop(0, chunks, chunk, 0)

        plsc.subcore_barrier()
        pltpu.sync_copy(acch.at[pl.ds(sid * rows_pt, rows_pt)],
                        outh_hbm.at[pl.ds(cid * np_ + sid * rows_pt, rows_pt)])
        pltpu.sync_copy(accw.at[pl.ds(sid * rows_pt, rows_pt)],
                        outw_hbm.at[pl.ds(cid * np_ + sid * rows_pt, rows_pt)])

    return pl.kernel(
        body,
        out_type=(jax.ShapeDtypeStruct((2 * np_, 128), jnp.float32),
                  jax.ShapeDtypeStruct((2 * np_, 16), jnp.float32)),
        compiler_params=pltpu.CompilerParams(use_tc_tiling_on_sc=False, needs_layout_passes=False),
        mesh=plsc.VectorSubcoreMesh(core_axis_name="c", subcore_axis_name="s"),
        scratch_types=[
            pltpu.VMEM_SHARED((np_, 128), jnp.float32),
            pltpu.VMEM_SHARED((np_, 16), jnp.float32),
            pltpu.VMEM((ZR, 128), jnp.float32),
            pltpu.VMEM((ZR, 16), jnp.float32),
            pltpu.VMEM((16,), jnp.float32),
            pltpu.VMEM((B,), jnp.int32),
            pltpu.VMEM((B,), jnp.int32),
            pltpu.VMEM((B, 16), jnp.float32),
            pltpu.VMEM((B, 16), jnp.float32),
            pltpu.VMEM((B, 16), jnp.float32),
            pltpu.VMEM((B, 128), jnp.float32),
            pltpu.SemaphoreType.DMA,
            pltpu.SemaphoreType.DMA,
            pltpu.SemaphoreType.DMA,
        ],
    )


def _sc_edge1(np_, chunks):
    rows_pt = np_ // 16

    def body(src_hbm, dst_hbm, a1_hbm, d1_hbm, h1e_hbm, g_hbm, out_hbm,
             accs, zbuf, gv, a1v, d1v, sbuf, dbuf, wbuf, msgbuf, s1, s3):
        cid = lax.axis_index("c")
        sid = lax.axis_index("s")
        z16 = jnp.zeros((16,), jnp.float32)

        def zrow(r, _):
            for j in range(5):
                zbuf[r, pl.ds(j * 16, 16)] = z16
            return 0
        lax.fori_loop(0, ZR, zrow, 0)
        for k in range(rows_pt // ZR):
            pltpu.sync_copy(zbuf, accs.at[pl.ds(sid * rows_pt + k * ZR, ZR)])
        plsc.subcore_barrier()

        pltpu.sync_copy(g_hbm, gv)
        pltpu.sync_copy(a1_hbm, a1v)
        pltpu.sync_copy(d1_hbm, d1v)
        g = gv[...]
        wid = sid * 2 + cid

        def chunk(k, _):
            base = (wid * chunks + k) * B
            pltpu.sync_copy(src_hbm.at[pl.ds(base, B)], sbuf)
            pltpu.sync_copy(dst_hbm.at[pl.ds(base, B)], dbuf)
            ch = pltpu.async_copy(h1e_hbm.at[sbuf], msgbuf, s3)

            def wcalc(q, _):
                srcv = sbuf[pl.ds(q * 16, 16)]
                dstv = dbuf[pl.ds(q * 16, 16)]
                a = plsc.load_gather(a1v, [srcv])
                d = plsc.load_gather(d1v, [dstv])
                zz = a + d
                lz = jnp.where(zz > 0, zz, 0.2 * zz)
                m = jnp.maximum(g + d, 0.0)
                wbuf[pl.ds(q * 16, 16)] = jnp.exp(lz - m)
                return 0
            lax.fori_loop(0, B // 16, wcalc, 0)
            ch.wait()

            def mcalc(q, _):
                wvec = wbuf[pl.ds(q * 16, 16)]
                for i in range(16):
                    e = q * 16 + i
                    ws = wvec[i]
                    for j in range(5):
                        msgbuf[e, pl.ds(j * 16, 16)] = ws * msgbuf[e, pl.ds(j * 16, 16)]
                return 0
            lax.fori_loop(0, B // 16, mcalc, 0)
            pltpu.sync_copy(msgbuf, accs.at[dbuf], add=True)
            return 0
        lax.fori_loop(0, chunks, chunk, 0)

        plsc.subcore_barrier()
        pltpu.sync_copy(accs.at[pl.ds(sid * rows_pt, rows_pt)],
                        out_hbm.at[pl.ds(cid * np_ + sid * rows_pt, rows_pt)])

    return pl.kernel(
        body,
        out_type=jax.ShapeDtypeStruct((2 * np_, 80), jnp.float32),
        compiler_params=pltpu.CompilerParams(use_tc_tiling_on_sc=False, needs_layout_passes=False),
        mesh=plsc.VectorSubcoreMesh(core_axis_name="c", subcore_axis_name="s"),
        scratch_types=[
            pltpu.VMEM_SHARED((np_, 80), jnp.float32),
            pltpu.VMEM((ZR, 80), jnp.float32),
            pltpu.VMEM((16,), jnp.float32),
            pltpu.VMEM((np_,), jnp.float32),
            pltpu.VMEM((np_,), jnp.float32),
            pltpu.VMEM((B,), jnp.int32),
            pltpu.VMEM((B,), jnp.int32),
            pltpu.VMEM((B,), jnp.float32),
            pltpu.VMEM((B, 80), jnp.float32),
            pltpu.SemaphoreType.DMA,
            pltpu.SemaphoreType.DMA,
        ],
    )


def kernel(x, edge_index, W0, a_src0, a_dst0, b0, W1, a_src1, a_dst1, b1):
    N, IN = x.shape
    HC = W0.shape[1]
    H = a_src0.shape[1]
    C = HC // H
    OC = W1.shape[1]
    f32 = jnp.float32

    np_ = ((N + 1023) // 1024) * 1024
    nblk = np_ // BLK

    ei = edge_index.astype(jnp.int32)
    loop = jnp.arange(N, dtype=jnp.int32)
    src = jnp.concatenate([ei[0], loop])
    dst = jnp.concatenate([ei[1], loop])
    etot = src.shape[0]
    step = 32 * B
    ep = ((etot + step - 1) // step) * step
    chunks = ep // step
    pad = ep - etot
    src = jnp.concatenate([src, jnp.zeros((pad,), jnp.int32)])
    dst = jnp.concatenate([dst, jnp.full((pad,), np_ - 1, jnp.int32)])

    xp = jnp.pad(x, ((0, np_ - N), (0, 0)))
    n_arr = jnp.array([N], jnp.int32)

    h0, U, V, g16 = pl.pallas_call(
        _tc_prep0,
        grid=(nblk,),
        in_specs=[
            pl.BlockSpec((BLK, IN), lambda i: (i, 0)),
            pl.BlockSpec((IN, HC), lambda i: (0, 0)),
            pl.BlockSpec((1, HC), lambda i: (0, 0)),
            pl.BlockSpec((1, HC), lambda i: (0, 0)),
            pl.BlockSpec(memory_space=pltpu.SMEM),
        ],
        out_specs=[
            pl.BlockSpec((BLK, HC), lambda i: (i, 0)),
            pl.BlockSpec((BLK, 16), lambda i: (i, 0)),
            pl.BlockSpec((BLK, 16), lambda i: (i, 0)),
            pl.BlockSpec((1, 16), lambda i: (0, 0)),
        ],
        out_shape=[
            jax.ShapeDtypeStruct((np_, HC), f32),
            jax.ShapeDtypeStruct((np_, 16), f32),
            jax.ShapeDtypeStruct((np_, 16), f32),
            jax.ShapeDtypeStruct((1, 16), f32),
        ],
        scratch_shapes=[pltpu.VMEM((8, 128), f32)],
    )(xp, W0, a_src0.reshape(1, HC), a_dst0.reshape(1, HC), n_arr)

    acch, accw = _sc_edge0(np_, chunks)(src, dst, U, V, h0, g16.reshape(16))

    h1e, adt, g1 = pl.pallas_call(
        _tc_mid,
        grid=(nblk,),
        in_specs=[
            pl.BlockSpec((BLK, HC), lambda i: (i, 0)),
            pl.BlockSpec((BLK, HC), lambda i, nb=nblk: (i + nb, 0)),
            pl.BlockSpec((BLK, 16), lambda i: (i, 0)),
            pl.BlockSpec((BLK, 16), lambda i, nb=nblk: (i + nb, 0)),
            pl.BlockSpec((1, HC), lambda i: (0, 0)),
            pl.BlockSpec((HC, OC), lambda i: (0, 0)),
            pl.BlockSpec((1, OC), lambda i: (0, 0)),
            pl.BlockSpec((1, OC), lambda i: (0, 0)),
            pl.BlockSpec(memory_space=pltpu.SMEM),
        ],
        out_specs=[
            pl.BlockSpec((BLK, 80), lambda i: (i, 0)),
            pl.BlockSpec((BLK, 16), lambda i: (i, 0)),
            pl.BlockSpec((1, 16), lambda i: (0, 0)),
        ],
        out_shape=[
            jax.ShapeDtypeStruct((np_, 80), f32),
            jax.ShapeDtypeStruct((np_, 16), f32),
            jax.ShapeDtypeStruct((1, 16), f32),
        ],
        scratch_shapes=[pltpu.VMEM((8, 128), f32)],
    )(acch, acch, accw, accw, b0.reshape(1, HC), W1, a_src1.reshape(1, OC),
      a_dst1.reshape(1, OC), n_arr)

    a1t = adt[:, 0].reshape(np_)
    d1t = adt[:, 1].reshape(np_)
    acc1 = _sc_edge1(np_, chunks)(src, dst, a1t, d1t, h1e, g1.reshape(16))

    out = pl.pallas_call(
        _tc_final,
        grid=(nblk,),
        in_specs=[
            pl.BlockSpec((BLK, 80), lambda i: (i, 0)),
            pl.BlockSpec((BLK, 80), lambda i, nb=nblk: (i + nb, 0)),
            pl.BlockSpec((1, OC), lambda i: (0, 0)),
        ],
        out_specs=pl.BlockSpec((BLK, OC), lambda i: (i, 0)),
        out_shape=jax.ShapeDtypeStruct((np_, OC), f32),
    )(acc1, acc1, b1.reshape(1, OC))

    return out[:N]


# speedup vs baseline: 60.9354x; 1.0579x over previous
import functools

import jax
import jax.numpy as jnp
from jax import lax
from jax.experimental import pallas as pl
from jax.experimental.pallas import tpu as pltpu
from jax.experimental.pallas import tpu_sc as plsc

NEG = -1.0e30
BIG = 1.0e30
B = 128
ZR = 32
BLK = 512


def _tc_prep0(x_ref, w0_ref, as_ref, ad_ref, n_ref, h0_o, u_o, v_o, g_o, gsc):
    i = pl.program_id(0)
    n_real = n_ref[0]
    h0 = jnp.dot(x_ref[...], w0_ref[...], preferred_element_type=jnp.float32)
    h0_o[...] = h0
    ps = h0 * as_ref[...]
    pd = h0 * ad_ref[...]
    H = u_o.shape[1] // 2
    C = h0.shape[1] // H
    asrc = jnp.concatenate(
        [jnp.sum(ps[:, h * C:(h + 1) * C], axis=1, keepdims=True) for h in range(H)], axis=1)
    adst = jnp.concatenate(
        [jnp.sum(pd[:, h * C:(h + 1) * C], axis=1, keepdims=True) for h in range(H)], axis=1)
    rowid = i * BLK + lax.broadcasted_iota(jnp.int32, (BLK, 1), 0)
    valid = rowid < n_real
    asrc = jnp.where(valid, asrc, NEG)
    adst = jnp.where(valid, adst, NEG)
    z8 = jnp.zeros((BLK, H), jnp.float32)
    u_o[...] = jnp.concatenate([asrc, z8], axis=1)
    v_o[...] = jnp.concatenate([adst, z8], axis=1)

    @pl.when(i == 0)
    def _():
        gsc[...] = jnp.full((8, 128), NEG, jnp.float32)

    bm = jnp.max(asrc, axis=0, keepdims=True)
    gsc[0:1, 0:H] = jnp.maximum(gsc[0:1, 0:H], bm)
    g_o[...] = jnp.concatenate(
        [gsc[0:1, 0:H], jnp.full((1, 16 - H), BIG, jnp.float32)], axis=1)


def _tc_mid(a0_ref, a1_ref, wa0_ref, wa1_ref, b0_ref, w1_ref, as1_ref, ad1_ref,
            n_ref, h1e_o, adt_o, g1_o, gsc):
    i = pl.program_id(0)
    n_real = n_ref[0]
    s = a0_ref[...] + a1_ref[...]
    den = wa0_ref[...] + wa1_ref[...]
    HC = b0_ref.shape[1]
    H = wa0_ref.shape[1] // 2
    C = HC // H
    outs = []
    for h in range(H):
        outs.append(s[:, h * C:(h + 1) * C] / (den[:, h:h + 1] + 1e-16))
    z = jnp.concatenate(outs, axis=1) + b0_ref[...]
    hp = jnp.where(z > 0, z, jnp.exp(jnp.minimum(z, 0.0)) - 1.0)
    h1 = jnp.dot(hp, w1_ref[...], preferred_element_type=jnp.float32)
    as1 = jnp.sum(h1 * as1_ref[...], axis=1, keepdims=True)
    ad1 = jnp.sum(h1 * ad1_ref[...], axis=1, keepdims=True)
    rowid = i * BLK + lax.broadcasted_iota(jnp.int32, (BLK, 1), 0)
    valid = rowid < n_real
    as1 = jnp.where(valid, as1, NEG)
    ad1 = jnp.where(valid, ad1, NEG)
    OC = h1.shape[1]
    h1e_o[...] = jnp.concatenate(
        [h1, jnp.ones((BLK, 1), jnp.float32), jnp.zeros((BLK, 15), jnp.float32)], axis=1)
    adt_o[...] = jnp.concatenate(
        [as1, ad1, jnp.zeros((BLK, 14), jnp.float32)], axis=1)

    @pl.when(i == 0)
    def _():
        gsc[...] = jnp.full((8, 128), NEG, jnp.float32)

    gsc[0:1, 0:1] = jnp.maximum(gsc[0:1, 0:1], jnp.max(as1, axis=0, keepdims=True))
    g1_o[...] = jnp.broadcast_to(gsc[0:1, 0:1], (1, 16))


def _tc_final(a0_ref, a1_ref, b1_ref, out_o):
    s = a0_ref[...] + a1_ref[...]
    OC = b1_ref.shape[1]
    out_o[...] = s[:, 0:OC] / (s[:, OC:OC + 1] + 1e-16) + b1_ref[...]


def _sc_edge0(np_, chunks):
    rows_pt = np_ // 16

    def body(src_hbm, dst_hbm, u_hbm, v_hbm, h0_hbm, g_hbm, outh_hbm, outw_hbm,
             acch, accw, zbuf, zwbuf, gv, sbuf, dbuf, ubuf, vbuf, wbuf, hbuf,
             s1, s2, s3):
        cid = lax.axis_index("c")
        sid = lax.axis_index("s")
        z16 = jnp.zeros((16,), jnp.float32)

        def zrow(r, _):
            for j in range(8):
                zbuf[r, pl.ds(j * 16, 16)] = z16
            zwbuf[r] = z16
            return 0
        lax.fori_loop(0, ZR, zrow, 0)

        def zcopy(k, _):
            pltpu.sync_copy(zbuf, acch.at[pl.ds(sid * rows_pt + k * ZR, ZR)])
            pltpu.sync_copy(zwbuf, accw.at[pl.ds(sid * rows_pt + k * ZR, ZR)])
            return 0
        lax.fori_loop(0, rows_pt // ZR, zcopy, 0)
        plsc.subcore_barrier()

        pltpu.sync_copy(g_hbm, gv)
        g = gv[...]
        wid = sid * 2 + cid

        def chunk(k, _):
            base = (wid * chunks + k) * B
            pltpu.sync_copy(src_hbm.at[pl.ds(base, B)], sbuf)
            pltpu.sync_copy(dst_hbm.at[pl.ds(base, B)], dbuf)
            cu = pltpu.async_copy(u_hbm.at[sbuf], ubuf, s1)
            cv = pltpu.async_copy(v_hbm.at[dbuf], vbuf, s2)
            ch = pltpu.async_copy(h0_hbm.at[sbuf], hbuf, s3)
            cu.wait()
            cv.wait()

            @plsc.parallel_loop(0, B, unroll=4)
            def wcalc(e):
                uz = ubuf[e]
                vz = vbuf[e]
                zz = uz + vz
                lz = jnp.where(zz > 0, zz, 0.2 * zz)
                m = jnp.maximum(g + vz, 0.0)
                wbuf[e] = jnp.exp(lz - m)
            ch.wait()

            @plsc.parallel_loop(0, B, unroll=2)
            def mcalc(e):
                wvec = wbuf[e]
                for j in range(8):
                    ws = wvec[j]
                    hbuf[e, pl.ds(j * 16, 16)] = ws * hbuf[e, pl.ds(j * 16, 16)]
            pltpu.sync_copy(hbuf, acch.at[dbuf], add=True)
            pltpu.sync_copy(wbuf, accw.at[dbuf], add=True)
            return 0
        lax.fori_loop(0, chunks, chunk, 0)

        plsc.subcore_barrier()
        pltpu.sync_copy(acch.at[pl.ds(sid * rows_pt, rows_pt)],
                        outh_hbm.at[pl.ds(cid * np_ + sid * rows_pt, rows_pt)])
        pltpu.sync_copy(accw.at[pl.ds(sid * rows_pt, rows_pt)],
                        outw_hbm.at[pl.ds(cid * np_ + sid * rows_pt, rows_pt)])

    return pl.kernel(
        body,
        out_type=(jax.ShapeDtypeStruct((2 * np_, 128), jnp.float32),
                  jax.ShapeDtypeStruct((2 * np_, 16), jnp.float32)),
        compiler_params=pltpu.CompilerParams(use_tc_tiling_on_sc=False, needs_layout_passes=False),
        mesh=plsc.VectorSubcoreMesh(core_axis_name="c", subcore_axis_name="s"),
        scratch_types=[
            pltpu.VMEM_SHARED((np_, 128), jnp.float32),
            pltpu.VMEM_SHARED((np_, 16), jnp.float32),
            pltpu.VMEM((ZR, 128), jnp.float32),
            pltpu.VMEM((ZR, 16), jnp.float32),
            pltpu.VMEM((16,), jnp.float32),
            pltpu.VMEM((B,), jnp.int32),
            pltpu.VMEM((B,), jnp.int32),
            pltpu.VMEM((B, 16), jnp.float32),
            pltpu.VMEM((B, 16), jnp.float32),
            pltpu.VMEM((B, 16), jnp.float32),
            pltpu.VMEM((B, 128), jnp.float32),
            pltpu.SemaphoreType.DMA,
            pltpu.SemaphoreType.DMA,
            pltpu.SemaphoreType.DMA,
        ],
    )


def _sc_edge1(np_, chunks):
    rows_pt = np_ // 16

    def body(src_hbm, dst_hbm, a1_hbm, d1_hbm, h1e_hbm, g_hbm, out_hbm,
             accs, zbuf, gv, a1v, d1v, sbuf, dbuf, wbuf, msgbuf, s1, s3):
        cid = lax.axis_index("c")
        sid = lax.axis_index("s")
        z16 = jnp.zeros((16,), jnp.float32)

        def zrow(r, _):
            for j in range(5):
                zbuf[r, pl.ds(j * 16, 16)] = z16
            return 0
        lax.fori_loop(0, ZR, zrow, 0)
        for k in range(rows_pt // ZR):
            pltpu.sync_copy(zbuf, accs.at[pl.ds(sid * rows_pt + k * ZR, ZR)])
        plsc.subcore_barrier()

        pltpu.sync_copy(g_hbm, gv)
        pltpu.sync_copy(a1_hbm, a1v)
        pltpu.sync_copy(d1_hbm, d1v)
        g = gv[...]
        wid = sid * 2 + cid

        def chunk(k, _):
            base = (wid * chunks + k) * B
            pltpu.sync_copy(src_hbm.at[pl.ds(base, B)], sbuf)
            pltpu.sync_copy(dst_hbm.at[pl.ds(base, B)], dbuf)
            ch = pltpu.async_copy(h1e_hbm.at[sbuf], msgbuf, s3)

            @plsc.parallel_loop(0, B // 16, unroll=2)
            def wcalc(q):
                srcv = sbuf[pl.ds(q * 16, 16)]
                dstv = dbuf[pl.ds(q * 16, 16)]
                a = plsc.load_gather(a1v, [srcv])
                d = plsc.load_gather(d1v, [dstv])
                zz = a + d
                lz = jnp.where(zz > 0, zz, 0.2 * zz)
                m = jnp.maximum(g + d, 0.0)
                wbuf[pl.ds(q * 16, 16)] = jnp.exp(lz - m)
            ch.wait()

            @plsc.parallel_loop(0, B // 16)
            def mcalc(q):
                wvec = wbuf[pl.ds(q * 16, 16)]
                for i in range(16):
                    e = q * 16 + i
                    ws = wvec[i]
                    for j in range(5):
                        msgbuf[e, pl.ds(j * 16, 16)] = ws * msgbuf[e, pl.ds(j * 16, 16)]
            pltpu.sync_copy(msgbuf, accs.at[dbuf], add=True)
            return 0
        lax.fori_loop(0, chunks, chunk, 0)

        plsc.subcore_barrier()
        pltpu.sync_copy(accs.at[pl.ds(sid * rows_pt, rows_pt)],
                        out_hbm.at[pl.ds(cid * np_ + sid * rows_pt, rows_pt)])

    return pl.kernel(
        body,
        out_type=jax.ShapeDtypeStruct((2 * np_, 80), jnp.float32),
        compiler_params=pltpu.CompilerParams(use_tc_tiling_on_sc=False, needs_layout_passes=False),
        mesh=plsc.VectorSubcoreMesh(core_axis_name="c", subcore_axis_name="s"),
        scratch_types=[
            pltpu.VMEM_SHARED((np_, 80), jnp.float32),
            pltpu.VMEM((ZR, 80), jnp.float32),
            pltpu.VMEM((16,), jnp.float32),
            pltpu.VMEM((np_,), jnp.float32),
            pltpu.VMEM((np_,), jnp.float32),
            pltpu.VMEM((B,), jnp.int32),
            pltpu.VMEM((B,), jnp.int32),
            pltpu.VMEM((B,), jnp.float32),
            pltpu.VMEM((B, 80), jnp.float32),
            pltpu.SemaphoreType.DMA,
            pltpu.SemaphoreType.DMA,
        ],
    )


def kernel(x, edge_index, W0, a_src0, a_dst0, b0, W1, a_src1, a_dst1, b1):
    N, IN = x.shape
    HC = W0.shape[1]
    H = a_src0.shape[1]
    C = HC // H
    OC = W1.shape[1]
    f32 = jnp.float32

    np_ = ((N + 1023) // 1024) * 1024
    nblk = np_ // BLK

    ei = edge_index.astype(jnp.int32)
    loop = jnp.arange(N, dtype=jnp.int32)
    src = jnp.concatenate([ei[0], loop])
    dst = jnp.concatenate([ei[1], loop])
    etot = src.shape[0]
    step = 32 * B
    ep = ((etot + step - 1) // step) * step
    chunks = ep // step
    pad = ep - etot
    src = jnp.concatenate([src, jnp.zeros((pad,), jnp.int32)])
    dst = jnp.concatenate([dst, jnp.full((pad,), np_ - 1, jnp.int32)])

    xp = jnp.pad(x, ((0, np_ - N), (0, 0)))
    n_arr = jnp.array([N], jnp.int32)

    h0, U, V, g16 = pl.pallas_call(
        _tc_prep0,
        grid=(nblk,),
        in_specs=[
            pl.BlockSpec((BLK, IN), lambda i: (i, 0)),
            pl.BlockSpec((IN, HC), lambda i: (0, 0)),
            pl.BlockSpec((1, HC), lambda i: (0, 0)),
            pl.BlockSpec((1, HC), lambda i: (0, 0)),
            pl.BlockSpec(memory_space=pltpu.SMEM),
        ],
        out_specs=[
            pl.BlockSpec((BLK, HC), lambda i: (i, 0)),
            pl.BlockSpec((BLK, 16), lambda i: (i, 0)),
            pl.BlockSpec((BLK, 16), lambda i: (i, 0)),
            pl.BlockSpec((1, 16), lambda i: (0, 0)),
        ],
        out_shape=[
            jax.ShapeDtypeStruct((np_, HC), f32),
            jax.ShapeDtypeStruct((np_, 16), f32),
            jax.ShapeDtypeStruct((np_, 16), f32),
            jax.ShapeDtypeStruct((1, 16), f32),
        ],
        scratch_shapes=[pltpu.VMEM((8, 128), f32)],
    )(xp, W0, a_src0.reshape(1, HC), a_dst0.reshape(1, HC), n_arr)

    acch, accw = _sc_edge0(np_, chunks)(src, dst, U, V, h0, g16.reshape(16))

    h1e, adt, g1 = pl.pallas_call(
        _tc_mid,
        grid=(nblk,),
        in_specs=[
            pl.BlockSpec((BLK, HC), lambda i: (i, 0)),
            pl.BlockSpec((BLK, HC), lambda i, nb=nblk: (i + nb, 0)),
            pl.BlockSpec((BLK, 16), lambda i: (i, 0)),
            pl.BlockSpec((BLK, 16), lambda i, nb=nblk: (i + nb, 0)),
            pl.BlockSpec((1, HC), lambda i: (0, 0)),
            pl.BlockSpec((HC, OC), lambda i: (0, 0)),
            pl.BlockSpec((1, OC), lambda i: (0, 0)),
            pl.BlockSpec((1, OC), lambda i: (0, 0)),
            pl.BlockSpec(memory_space=pltpu.SMEM),
        ],
        out_specs=[
            pl.BlockSpec((BLK, 80), lambda i: (i, 0)),
            pl.BlockSpec((BLK, 16), lambda i: (i, 0)),
            pl.BlockSpec((1, 16), lambda i: (0, 0)),
        ],
        out_shape=[
            jax.ShapeDtypeStruct((np_, 80), f32),
            jax.ShapeDtypeStruct((np_, 16), f32),
            jax.ShapeDtypeStruct((1, 16), f32),
        ],
        scratch_shapes=[pltpu.VMEM((8, 128), f32)],
    )(acch, acch, accw, accw, b0.reshape(1, HC), W1, a_src1.reshape(1, OC),
      a_dst1.reshape(1, OC), n_arr)

    a1t = adt[:, 0].reshape(np_)
    d1t = adt[:, 1].reshape(np_)
    acc1 = _sc_edge1(np_, chunks)(src, dst, a1t, d1t, h1e, g1.reshape(16))

    out = pl.pallas_call(
        _tc_final,
        grid=(nblk,),
        in_specs=[
            pl.BlockSpec((BLK, 80), lambda i: (i, 0)),
            pl.BlockSpec((BLK, 80), lambda i, nb=nblk: (i + nb, 0)),
            pl.BlockSpec((1, OC), lambda i: (0, 0)),
        ],
        out_specs=pl.BlockSpec((BLK, OC), lambda i: (i, 0)),
        out_shape=jax.ShapeDtypeStruct((np_, OC), f32),
    )(acc1, acc1, b1.reshape(1, OC))

    return out[:N]


# speedup vs baseline: 61.0676x; 1.0022x over previous
import jax
import jax.numpy as jnp
from jax import lax
from jax.experimental import pallas as pl
from jax.experimental.pallas import tpu as pltpu
from jax.experimental.pallas import tpu_sc as plsc

NEG = -1.0e30
BIG = 1.0e30
B = 128
BLK = 512
BLK2 = 1000


def _tc_prep0(x_ref, w0_ref, as_ref, ad_ref, n_ref, h0_o, u_o, v_o, g_o, gsc):
    i = pl.program_id(0)
    blk = x_ref.shape[0]
    n_real = n_ref[0]
    h0 = jnp.dot(x_ref[...], w0_ref[...], preferred_element_type=jnp.float32)
    h0_o[...] = h0
    ps = h0 * as_ref[...]
    pd = h0 * ad_ref[...]
    H = u_o.shape[1] // 2
    C = h0.shape[1] // H
    asrc = jnp.concatenate(
        [jnp.sum(ps[:, h * C:(h + 1) * C], axis=1, keepdims=True) for h in range(H)], axis=1)
    adst = jnp.concatenate(
        [jnp.sum(pd[:, h * C:(h + 1) * C], axis=1, keepdims=True) for h in range(H)], axis=1)
    rowid = i * blk + lax.broadcasted_iota(jnp.int32, (blk, 1), 0)
    valid = rowid < n_real
    asrc = jnp.where(valid, asrc, NEG)
    adst = jnp.where(valid, adst, NEG)
    z8 = jnp.zeros((blk, H), jnp.float32)
    u_o[...] = jnp.concatenate([asrc, z8], axis=1)
    v_o[...] = jnp.concatenate([adst, z8], axis=1)

    @pl.when(i == 0)
    def _():
        gsc[...] = jnp.full((8, 128), NEG, jnp.float32)

    bm = jnp.max(asrc, axis=0, keepdims=True)
    gsc[0:1, 0:H] = jnp.maximum(gsc[0:1, 0:H], bm)
    g_o[...] = jnp.concatenate(
        [gsc[0:1, 0:H], jnp.full((1, 16 - H), BIG, jnp.float32)], axis=1)


def _tc_mid(a0_ref, a1_ref, wa0_ref, wa1_ref, b0_ref, w1_ref, as1_ref, ad1_ref,
            h1e_o, adt_o, g1_o, gsc):
    i = pl.program_id(0)
    blk = a0_ref.shape[0]
    s = a0_ref[...] + a1_ref[...]
    den = wa0_ref[...] + wa1_ref[...]
    HC = b0_ref.shape[1]
    H = wa0_ref.shape[1] // 2
    C = HC // H
    outs = []
    for h in range(H):
        outs.append(s[:, h * C:(h + 1) * C] / (den[:, h:h + 1] + 1e-16))
    z = jnp.concatenate(outs, axis=1) + b0_ref[...]
    hp = jnp.where(z > 0, z, jnp.exp(jnp.minimum(z, 0.0)) - 1.0)
    h1 = jnp.dot(hp, w1_ref[...], preferred_element_type=jnp.float32)
    as1 = jnp.sum(h1 * as1_ref[...], axis=1, keepdims=True)
    ad1 = jnp.sum(h1 * ad1_ref[...], axis=1, keepdims=True)
    h1e_o[...] = jnp.concatenate(
        [h1, jnp.ones((blk, 1), jnp.float32), jnp.zeros((blk, 15), jnp.float32)], axis=1)
    adt_o[...] = jnp.concatenate(
        [as1, ad1, jnp.zeros((blk, 14), jnp.float32)], axis=1)

    @pl.when(i == 0)
    def _():
        gsc[...] = jnp.full((8, 128), NEG, jnp.float32)

    gsc[0:1, 0:1] = jnp.maximum(gsc[0:1, 0:1], jnp.max(as1, axis=0, keepdims=True))
    g1_o[...] = jnp.broadcast_to(gsc[0:1, 0:1], (1, 16))


def _tc_final(a0_ref, a1_ref, b1_ref, out_o):
    s = a0_ref[...] + a1_ref[...]
    OC = b1_ref.shape[1]
    out_o[...] = s[:, 0:OC] / (s[:, OC:OC + 1] + 1e-16) + b1_ref[...]


def _sc_edge0(n, chunks):
    rows_pt = n // 16

    def body(src_hbm, dst_hbm, u_hbm, v_hbm, h0_hbm, g_hbm, z128_hbm, z16_hbm,
             outh_hbm, outw_hbm,
             acch, accw, gv, sbuf0, dbuf0, sbuf1, dbuf1, ubuf, vbuf, wbuf,
             hbuf0, hbuf1, suv, sh0, sh1):
        cid = lax.axis_index("c")
        sid = lax.axis_index("s")
        wid = sid * 2 + cid
        ebase = wid * chunks
        r0 = sid * rows_pt

        pltpu.sync_copy(g_hbm, gv)
        pltpu.sync_copy(src_hbm.at[pl.ds(ebase * B, B)], sbuf0)
        pltpu.sync_copy(dst_hbm.at[pl.ds(ebase * B, B)], dbuf0)
        pltpu.sync_copy(src_hbm.at[pl.ds((ebase + 1) * B, B)], sbuf1)
        pltpu.sync_copy(dst_hbm.at[pl.ds((ebase + 1) * B, B)], dbuf1)
        pltpu.async_copy(u_hbm.at[sbuf0], ubuf, suv)
        pltpu.async_copy(v_hbm.at[dbuf0], vbuf, suv)
        pltpu.async_copy(h0_hbm.at[sbuf0], hbuf0, sh0)
        pltpu.async_copy(h0_hbm.at[sbuf1], hbuf1, sh1)

        pltpu.sync_copy(z128_hbm.at[pl.ds(r0, rows_pt)], acch.at[pl.ds(r0, rows_pt)])
        pltpu.sync_copy(z16_hbm.at[pl.ds(r0, rows_pt)], accw.at[pl.ds(r0, rows_pt)])
        plsc.subcore_barrier()

        g = gv[...]

        def phase(c, sb, db, hb, sh, sb_o, db_o):
            pltpu.make_async_copy(u_hbm.at[sb], ubuf, suv).wait()
            pltpu.make_async_copy(v_hbm.at[db], vbuf, suv).wait()

            @plsc.parallel_loop(0, B, unroll=4)
            def wcalc(e):
                uz = ubuf[e]
                vz = vbuf[e]
                zz = uz + vz
                lz = jnp.where(zz > 0, zz, 0.2 * zz)
                m = jnp.maximum(g + vz, 0.0)
                wbuf[e] = jnp.exp(lz - m)

            @pl.when(c + 1 < chunks)
            def _():
                pltpu.async_copy(u_hbm.at[sb_o], ubuf, suv)
                pltpu.async_copy(v_hbm.at[db_o], vbuf, suv)

            pltpu.make_async_copy(h0_hbm.at[sb], hb, sh).wait()

            @plsc.parallel_loop(0, B, unroll=2)
            def mcalc(e):
                wvec = wbuf[e]
                for j in range(8):
                    hb[e, pl.ds(j * 16, 16)] = wvec[j] * hb[e, pl.ds(j * 16, 16)]

            pltpu.sync_copy(hb, acch.at[db], add=True)
            pltpu.sync_copy(wbuf, accw.at[db], add=True)

            @pl.when(c + 2 < chunks)
            def _():
                pltpu.sync_copy(src_hbm.at[pl.ds((ebase + c + 2) * B, B)], sb)
                pltpu.sync_copy(dst_hbm.at[pl.ds((ebase + c + 2) * B, B)], db)
                pltpu.async_copy(h0_hbm.at[sb], hb, sh)

        def pair(p, _):
            c0 = 2 * p
            phase(c0, sbuf0, dbuf0, hbuf0, sh0, sbuf1, dbuf1)
            phase(c0 + 1, sbuf1, dbuf1, hbuf1, sh1, sbuf0, dbuf0)
            return 0
        lax.fori_loop(0, chunks // 2, pair, 0)

        plsc.subcore_barrier()
        pltpu.sync_copy(acch.at[pl.ds(r0, rows_pt)],
                        outh_hbm.at[pl.ds(cid * n + r0, rows_pt)])
        pltpu.sync_copy(accw.at[pl.ds(r0, rows_pt)],
                        outw_hbm.at[pl.ds(cid * n + r0, rows_pt)])

    return pl.kernel(
        body,
        out_type=(jax.ShapeDtypeStruct((2 * n, 128), jnp.float32),
                  jax.ShapeDtypeStruct((2 * n, 16), jnp.float32)),
        compiler_params=pltpu.CompilerParams(
            use_tc_tiling_on_sc=False, needs_layout_passes=False),
        mesh=plsc.VectorSubcoreMesh(core_axis_name="c", subcore_axis_name="s"),
        scratch_types=[
            pltpu.VMEM_SHARED((n, 128), jnp.float32),
            pltpu.VMEM_SHARED((n, 16), jnp.float32),
            pltpu.VMEM((16,), jnp.float32),
            pltpu.VMEM((B,), jnp.int32),
            pltpu.VMEM((B,), jnp.int32),
            pltpu.VMEM((B,), jnp.int32),
            pltpu.VMEM((B,), jnp.int32),
            pltpu.VMEM((B, 16), jnp.float32),
            pltpu.VMEM((B, 16), jnp.float32),
            pltpu.VMEM((B, 16), jnp.float32),
            pltpu.VMEM((B, 128), jnp.float32),
            pltpu.VMEM((B, 128), jnp.float32),
            pltpu.SemaphoreType.DMA,
            pltpu.SemaphoreType.DMA,
            pltpu.SemaphoreType.DMA,
        ],
    )


def _sc_edge1(n, np_, chunks):
    rows_pt = n // 16

    def body(src_hbm, dst_hbm, a1_hbm, d1_hbm, h1e_hbm, g_hbm, z80_hbm, out_hbm,
             accs, gv, a1v, d1v, sbuf0, dbuf0, sbuf1, dbuf1, wbuf,
             mbuf0, mbuf1, sh0, sh1):
        cid = lax.axis_index("c")
        sid = lax.axis_index("s")
        wid = sid * 2 + cid
        ebase = wid * chunks
        r0 = sid * rows_pt

        pltpu.sync_copy(g_hbm, gv)
        pltpu.sync_copy(src_hbm.at[pl.ds(ebase * B, B)], sbuf0)
        pltpu.sync_copy(dst_hbm.at[pl.ds(ebase * B, B)], dbuf0)
        pltpu.sync_copy(src_hbm.at[pl.ds((ebase + 1) * B, B)], sbuf1)
        pltpu.sync_copy(dst_hbm.at[pl.ds((ebase + 1) * B, B)], dbuf1)
        pltpu.async_copy(h1e_hbm.at[sbuf0], mbuf0, sh0)
        pltpu.async_copy(h1e_hbm.at[sbuf1], mbuf1, sh1)
        pltpu.sync_copy(a1_hbm, a1v)
        pltpu.sync_copy(d1_hbm, d1v)

        pltpu.sync_copy(z80_hbm.at[pl.ds(r0, rows_pt)], accs.at[pl.ds(r0, rows_pt)])
        plsc.subcore_barrier()

        g = gv[...]

        def phase(c, sb, db, mb, sh):
            @plsc.parallel_loop(0, B // 16, unroll=2)
            def wcalc(q):
                srcv = sb[pl.ds(q * 16, 16)]
                dstv = db[pl.ds(q * 16, 16)]
                a = plsc.load_gather(a1v, [srcv])
                d = plsc.load_gather(d1v, [dstv])
                zz = a + d
                lz = jnp.where(zz > 0, zz, 0.2 * zz)
                m = jnp.maximum(g + d, 0.0)
                wbuf[pl.ds(q * 16, 16)] = jnp.exp(lz - m)

            pltpu.make_async_copy(h1e_hbm.at[sb], mb, sh).wait()

            @plsc.parallel_loop(0, B // 16)
            def mcalc(q):
                wvec = wbuf[pl.ds(q * 16, 16)]
                for i in range(16):
                    e = q * 16 + i
                    ws = wvec[i]
                    for j in range(5):
                        mb[e, pl.ds(j * 16, 16)] = ws * mb[e, pl.ds(j * 16, 16)]

            pltpu.sync_copy(mb, accs.at[db], add=True)

            @pl.when(c + 2 < chunks)
            def _():
                pltpu.sync_copy(src_hbm.at[pl.ds((ebase + c + 2) * B, B)], sb)
                pltpu.sync_copy(dst_hbm.at[pl.ds((ebase + c + 2) * B, B)], db)
                pltpu.async_copy(h1e_hbm.at[sb], mb, sh)

        def pair(p, _):
            c0 = 2 * p
            phase(c0, sbuf0, dbuf0, mbuf0, sh0)
            phase(c0 + 1, sbuf1, dbuf1, mbuf1, sh1)
            return 0
        lax.fori_loop(0, chunks // 2, pair, 0)

        plsc.subcore_barrier()
        pltpu.sync_copy(accs.at[pl.ds(r0, rows_pt)],
                        out_hbm.at[pl.ds(cid * n + r0, rows_pt)])

    return pl.kernel(
        body,
        out_type=jax.ShapeDtypeStruct((2 * n, 80), jnp.float32),
        compiler_params=pltpu.CompilerParams(
            use_tc_tiling_on_sc=False, needs_layout_passes=False),
        mesh=plsc.VectorSubcoreMesh(core_axis_name="c", subcore_axis_name="s"),
        scratch_types=[
            pltpu.VMEM_SHARED((n, 80), jnp.float32),
            pltpu.VMEM((16,), jnp.float32),
            pltpu.VMEM((np_,), jnp.float32),
            pltpu.VMEM((np_,), jnp.float32),
            pltpu.VMEM((B,), jnp.int32),
            pltpu.VMEM((B,), jnp.int32),
            pltpu.VMEM((B,), jnp.int32),
            pltpu.VMEM((B,), jnp.int32),
            pltpu.VMEM((B,), jnp.float32),
            pltpu.VMEM((B, 80), jnp.float32),
            pltpu.VMEM((B, 80), jnp.float32),
            pltpu.SemaphoreType.DMA,
            pltpu.SemaphoreType.DMA,
        ],
    )


def kernel(x, edge_index, W0, a_src0, a_dst0, b0, W1, a_src1, a_dst1, b1):
    N, IN = x.shape
    HC = W0.shape[1]
    H = a_src0.shape[1]
    OC = W1.shape[1]
    f32 = jnp.float32

    np_ = ((N + 1023) // 1024) * 1024
    nblk = np_ // BLK
    nblk2 = N // BLK2

    ei = edge_index.astype(jnp.int32)
    loop = jnp.arange(N, dtype=jnp.int32)
    src = jnp.concatenate([ei[0], loop])
    dst = jnp.concatenate([ei[1], loop])
    etot = src.shape[0]
    step = 32 * B * 2
    ep = ((etot + step - 1) // step) * step
    chunks = ep // (32 * B)
    pad = ep - etot
    src = jnp.concatenate([src, jnp.full((pad,), np_ - 1, jnp.int32)])
    dst = jnp.concatenate([dst, jnp.zeros((pad,), jnp.int32)])

    xp = jnp.pad(x, ((0, np_ - N), (0, 0)))
    n_arr = jnp.array([N], jnp.int32)
    z128 = jnp.zeros((N, 128), f32)
    z16 = jnp.zeros((N, 16), f32)
    z80 = jnp.zeros((N, 80), f32)

    h0, U, V, g16 = pl.pallas_call(
        _tc_prep0,
        grid=(nblk,),
        in_specs=[
            pl.BlockSpec((BLK, IN), lambda i: (i, 0)),
            pl.BlockSpec((IN, HC), lambda i: (0, 0)),
            pl.BlockSpec((1, HC), lambda i: (0, 0)),
            pl.BlockSpec((1, HC), lambda i: (0, 0)),
            pl.BlockSpec(memory_space=pltpu.SMEM),
        ],
        out_specs=[
            pl.BlockSpec((BLK, HC), lambda i: (i, 0)),
            pl.BlockSpec((BLK, 16), lambda i: (i, 0)),
            pl.BlockSpec((BLK, 16), lambda i: (i, 0)),
            pl.BlockSpec((1, 16), lambda i: (0, 0)),
        ],
        out_shape=[
            jax.ShapeDtypeStruct((np_, HC), f32),
            jax.ShapeDtypeStruct((np_, 16), f32),
            jax.ShapeDtypeStruct((np_, 16), f32),
            jax.ShapeDtypeStruct((1, 16), f32),
        ],
        scratch_shapes=[pltpu.VMEM((8, 128), f32)],
    )(xp, W0, a_src0.reshape(1, HC), a_dst0.reshape(1, HC), n_arr)

    acch, accw = _sc_edge0(N, chunks)(
        src, dst, U, V, h0, g16.reshape(16), z128, z16)

    h1e, adt, g1 = pl.pallas_call(
        _tc_mid,
        grid=(nblk2,),
        in_specs=[
            pl.BlockSpec((BLK2, HC), lambda i: (i, 0)),
            pl.BlockSpec((BLK2, HC), lambda i, nb=nblk2: (i + nb, 0)),
            pl.BlockSpec((BLK2, 16), lambda i: (i, 0)),
            pl.BlockSpec((BLK2, 16), lambda i, nb=nblk2: (i + nb, 0)),
            pl.BlockSpec((1, HC), lambda i: (0, 0)),
            pl.BlockSpec((HC, OC), lambda i: (0, 0)),
            pl.BlockSpec((1, OC), lambda i: (0, 0)),
            pl.BlockSpec((1, OC), lambda i: (0, 0)),
        ],
        out_specs=[
            pl.BlockSpec((BLK2, 80), lambda i: (i, 0)),
            pl.BlockSpec((BLK2, 16), lambda i: (i, 0)),
            pl.BlockSpec((1, 16), lambda i: (0, 0)),
        ],
        out_shape=[
            jax.ShapeDtypeStruct((N, 80), f32),
            jax.ShapeDtypeStruct((N, 16), f32),
            jax.ShapeDtypeStruct((1, 16), f32),
        ],
        scratch_shapes=[pltpu.VMEM((8, 128), f32)],
    )(acch, acch, accw, accw, b0.reshape(1, HC), W1, a_src1.reshape(1, OC),
      a_dst1.reshape(1, OC))

    h1e_p = jnp.pad(h1e, ((0, np_ - N), (0, 0)))
    a1t = jnp.pad(adt[:, 0].reshape(N), (0, np_ - N), constant_values=NEG)
    d1t = jnp.pad(adt[:, 1].reshape(N), (0, np_ - N))
    acc1 = _sc_edge1(N, np_, chunks)(
        src, dst, a1t, d1t, h1e_p, g1.reshape(16), z80)

    out = pl.pallas_call(
        _tc_final,
        grid=(nblk2,),
        in_specs=[
            pl.BlockSpec((BLK2, 80), lambda i: (i, 0)),
            pl.BlockSpec((BLK2, 80), lambda i, nb=nblk2: (i + nb, 0)),
            pl.BlockSpec((1, OC), lambda i: (0, 0)),
        ],
        out_specs=pl.BlockSpec((BLK2, OC), lambda i: (i, 0)),
        out_shape=jax.ShapeDtypeStruct((N, OC), f32),
    )(acc1, acc1, b1.reshape(1, OC))

    return out
